# baseline (speedup 1.0000x reference)
import functools

import jax
import jax.numpy as jnp
from jax import lax
from jax.experimental import pallas as pl
from jax.experimental.pallas import tpu as pltpu
from jax.experimental.pallas import tpu_sc as plsc

N_NODES = 10000
D_FEAT = 128
N_EDGES = 320000

NC = 2
NS = 16
NW = NC * NS
N_PAD = 10240
RPT = N_PAD // NS
PAD_ROW = N_PAD - 8

CHUNK = 80
EDGES_PER_TILE = N_EDGES // NW
NCHUNK = EDGES_PER_TILE // CHUNK
HIST_W = 16

CH_S = 80
NCH_P = 25
NPIECE = 5
NBUF = 4

_mesh = plsc.VectorSubcoreMesh(
    core_axis_name="c", subcore_axis_name="s", num_cores=NC, num_subcores=NS)


def _fill_rows(buf, rows, width, value):
  v16 = jnp.full((16,), value, jnp.float32)

  def body(i, _):
    for g in range(width // 16):
      buf[i, pl.ds(g * 16, 16)] = v16
    return 0

  lax.fori_loop(0, rows, body, 0)


@functools.partial(
    pl.kernel,
    out_type=jax.ShapeDtypeStruct((NC, N_PAD, HIST_W), jnp.float32),
    mesh=_mesh,
    scratch_types=[
        pltpu.VMEM((NCHUNK, CHUNK), jnp.int32),
        pltpu.VMEM((CHUNK, HIST_W), jnp.float32),
        pltpu.VMEM_SHARED((N_PAD, HIST_W), jnp.float32),
        [pltpu.SemaphoreType.DMA for _ in range(4)],
    ],
    compiler_params=pltpu.CompilerParams(use_tc_tiling_on_sc=False),
)
def _hist_kernel(row_hbm, out_hbm, idx_v, ones_v, hist_sh, hsems):
  c = lax.axis_index("c")
  s = lax.axis_index("s")
  t = c * NS + s

  _fill_rows(ones_v, CHUNK, HIST_W, 0.0)
  for j in range(RPT // CHUNK):
    pltpu.sync_copy(ones_v, hist_sh.at[pl.ds(s * RPT + j * CHUNK, CHUNK)])
  _fill_rows(ones_v, CHUNK, HIST_W, 1.0)
  plsc.subcore_barrier()

  pltpu.sync_copy(row_hbm.at[t], idx_v)

  def body(j, _):
    for k in range(4):
      @pl.when(j % 4 == k)
      def _(k=k):
        @pl.when(j >= 4)
        def _():
          pltpu.make_async_copy(ones_v, hist_sh.at[idx_v.at[j]],
                                hsems[k]).wait()

        pltpu.async_copy(ones_v, hist_sh.at[idx_v.at[j]], hsems[k], add=True)

    return 0

  lax.fori_loop(0, NCHUNK, body, 0)
  for k in range(4):
    pltpu.make_async_copy(ones_v, hist_sh.at[idx_v.at[0]], hsems[k]).wait()
  plsc.subcore_barrier()

  pltpu.sync_copy(hist_sh.at[pl.ds(s * RPT, RPT)],
                  out_hbm.at[c, pl.ds(s * RPT, RPT)])


def _scale_body(wh_ref, f_ref, o_ref):
  rs = wh_ref[0] + wh_ref[1]
  d = jnp.where(rs > 0, lax.rsqrt(rs), jnp.zeros_like(rs))
  o_ref[...] = f_ref[...] * d[:, 0:1]


def _scale_features(wh, feats):
  n = feats.shape[0]
  bk = 1000
  grid = n // bk
  return pl.pallas_call(
      _scale_body,
      out_shape=jax.ShapeDtypeStruct(feats.shape, feats.dtype),
      grid=(grid,),
      in_specs=[
          pl.BlockSpec((NC, bk, HIST_W), lambda i: (0, i, 0)),
          pl.BlockSpec((bk, D_FEAT), lambda i: (i, 0)),
      ],
      out_specs=pl.BlockSpec((bk, D_FEAT), lambda i: (i, 0)),
  )(wh, feats)


@functools.partial(
    pl.kernel,
    out_type=jax.ShapeDtypeStruct((NC, N_PAD, D_FEAT), jnp.float32),
    mesh=_mesh,
    scratch_types=[
        pltpu.VMEM((NCH_P, CH_S), jnp.int32),
        pltpu.VMEM((NCH_P, CH_S), jnp.int32),
        [pltpu.VMEM((CH_S, D_FEAT), jnp.float32) for _ in range(NBUF)],
        pltpu.VMEM_SHARED((N_PAD, D_FEAT), jnp.float32),
        [pltpu.SemaphoreType.DMA for _ in range(NBUF)],
        [pltpu.SemaphoreType.DMA for _ in range(NBUF)],
    ],
    compiler_params=pltpu.CompilerParams(use_tc_tiling_on_sc=False),
)
def _spmm_kernel(feat_hbm, row_hbm, col_hbm, out_hbm,
                 rowi_v, coli_v, gbufs, acc_sh, gsems, ssems):
  gbuf0 = gbufs[0]
  c = lax.axis_index("c")
  s = lax.axis_index("s")
  t = c * NS + s

  _fill_rows(gbuf0, CH_S, D_FEAT, 0.0)
  for j in range(RPT // CH_S):
    pltpu.sync_copy(gbuf0, acc_sh.at[pl.ds(s * RPT + j * CH_S, CH_S)])
  plsc.subcore_barrier()

  for p in range(NPIECE):
    pltpu.sync_copy(row_hbm.at[t, p], rowi_v)
    pltpu.sync_copy(col_hbm.at[t, p], coli_v)

    pltpu.async_copy(feat_hbm.at[coli_v.at[0]], gbufs[0], gsems[0])
    pltpu.async_copy(feat_hbm.at[coli_v.at[1]], gbufs[1], gsems[1])

    def step(j, k):
      buf, gs, ss = gbufs[k], gsems[k], ssems[k]
      kn = (k + 2) % NBUF
      pltpu.make_async_copy(feat_hbm.at[coli_v.at[j]], buf, gs).wait()
      pltpu.async_copy(buf, acc_sh.at[rowi_v.at[j]], ss, add=True)

      @pl.when(j >= 2)
      def _():
        pltpu.make_async_copy(gbufs[kn], acc_sh.at[rowi_v.at[j]],
                              ssems[kn]).wait()

      @pl.when(j + 2 < NCH_P)
      def _():
        pltpu.async_copy(feat_hbm.at[coli_v.at[j + 2]], gbufs[kn], gsems[kn])

    def body(j, _):
      for k in range(NBUF):
        @pl.when(j % NBUF == k)
        def _(k=k):
          step(j, k)

      return 0

    lax.fori_loop(0, NCH_P, body, 0)

    pltpu.make_async_copy(gbufs[(NCH_P - 2) % NBUF],
                          acc_sh.at[rowi_v.at[0]],
                          ssems[(NCH_P - 2) % NBUF]).wait()
    pltpu.make_async_copy(gbufs[(NCH_P - 1) % NBUF],
                          acc_sh.at[rowi_v.at[0]],
                          ssems[(NCH_P - 1) % NBUF]).wait()

  plsc.subcore_barrier()

  pltpu.sync_copy(acc_sh.at[pl.ds(s * RPT, RPT)],
                  out_hbm.at[c, pl.ds(s * RPT, RPT)])


def _final_body(p_ref, wh_ref, o_ref):
  rs = wh_ref[0] + wh_ref[1]
  d = jnp.where(rs > 0, lax.rsqrt(rs), jnp.zeros_like(rs))
  o_ref[...] = (p_ref[0] + p_ref[1]) * d[:, 0:1]


def _final_combine(parts, wh):
  n = N_NODES
  bk = 1000
  grid = n // bk
  return pl.pallas_call(
      _final_body,
      out_shape=jax.ShapeDtypeStruct((n, D_FEAT), jnp.float32),
      grid=(grid,),
      in_specs=[
          pl.BlockSpec((NC, bk, D_FEAT), lambda i: (0, i, 0)),
          pl.BlockSpec((NC, bk, HIST_W), lambda i: (0, i, 0)),
      ],
      out_specs=pl.BlockSpec((bk, D_FEAT), lambda i: (i, 0)),
  )(parts, wh)


def kernel(features, edge_index):
  ei = edge_index.astype(jnp.int32)
  row3 = ei[0].reshape(NW, NCHUNK, CHUNK)
  row4 = ei[0].reshape(NW, NPIECE, NCH_P, CH_S)
  col4 = ei[1].reshape(NW, NPIECE, NCH_P, CH_S)
  wh = _hist_kernel(row3)
  fs = _scale_features(wh, features)
  parts = _spmm_kernel(fs, row4, col4)
  return _final_combine(parts, wh)


# speedup vs baseline: 1.0380x; 1.0380x over previous
import functools

import jax
import jax.numpy as jnp
from jax import lax
from jax.experimental import pallas as pl
from jax.experimental.pallas import tpu as pltpu
from jax.experimental.pallas import tpu_sc as plsc

N_NODES = 10000
D_FEAT = 128
N_EDGES = 320000

NC = 2
NS = 16
NW = NC * NS
N_PAD = 10240
RPT = N_PAD // NS
PAD_ROW = N_PAD - 8

CHUNK = 80
EDGES_PER_TILE = N_EDGES // NW
NCHUNK = EDGES_PER_TILE // CHUNK
HIST_W = 16

CH_S = 80
NCH_P = 25
NPIECE = 5
NBUF = 3

_mesh = plsc.VectorSubcoreMesh(
    core_axis_name="c", subcore_axis_name="s", num_cores=NC, num_subcores=NS)


def _fill_rows(buf, rows, width, value):
  v16 = jnp.full((16,), value, jnp.float32)

  def body(i, _):
    for g in range(width // 16):
      buf[i, pl.ds(g * 16, 16)] = v16
    return 0

  lax.fori_loop(0, rows, body, 0)


@functools.partial(
    pl.kernel,
    out_type=jax.ShapeDtypeStruct((NC, N_PAD, HIST_W), jnp.float32),
    mesh=_mesh,
    scratch_types=[
        pltpu.VMEM((NCHUNK, CHUNK), jnp.int32),
        pltpu.VMEM((CHUNK, HIST_W), jnp.float32),
        pltpu.VMEM_SHARED((N_PAD, HIST_W), jnp.float32),
    ],
    compiler_params=pltpu.CompilerParams(use_tc_tiling_on_sc=False),
)
def _hist_kernel(row_hbm, out_hbm, idx_v, ones_v, hist_sh):
  c = lax.axis_index("c")
  s = lax.axis_index("s")
  t = c * NS + s

  _fill_rows(ones_v, CHUNK, HIST_W, 0.0)
  for j in range(RPT // CHUNK):
    pltpu.sync_copy(ones_v, hist_sh.at[pl.ds(s * RPT + j * CHUNK, CHUNK)])
  _fill_rows(ones_v, CHUNK, HIST_W, 1.0)
  plsc.subcore_barrier()

  pltpu.sync_copy(row_hbm.at[t], idx_v)

  def body(j, _):
    pltpu.sync_copy(ones_v, hist_sh.at[idx_v.at[j]], add=True)
    return 0

  lax.fori_loop(0, NCHUNK, body, 0)
  plsc.subcore_barrier()

  pltpu.sync_copy(hist_sh.at[pl.ds(s * RPT, RPT)],
                  out_hbm.at[c, pl.ds(s * RPT, RPT)])


def _scale_body(wh_ref, f_ref, o_ref):
  rs = wh_ref[0] + wh_ref[1]
  d = jnp.where(rs > 0, lax.rsqrt(rs), jnp.zeros_like(rs))
  o_ref[...] = f_ref[...] * d[:, 0:1]


def _scale_features(wh, feats):
  n = feats.shape[0]
  bk = 1000
  grid = n // bk
  return pl.pallas_call(
      _scale_body,
      out_shape=jax.ShapeDtypeStruct(feats.shape, feats.dtype),
      grid=(grid,),
      in_specs=[
          pl.BlockSpec((NC, bk, HIST_W), lambda i: (0, i, 0)),
          pl.BlockSpec((bk, D_FEAT), lambda i: (i, 0)),
      ],
      out_specs=pl.BlockSpec((bk, D_FEAT), lambda i: (i, 0)),
  )(wh, feats)


@functools.partial(
    pl.kernel,
    out_type=jax.ShapeDtypeStruct((NC, N_PAD, D_FEAT), jnp.float32),
    mesh=_mesh,
    scratch_types=[
        pltpu.VMEM((NCH_P, CH_S), jnp.int32),
        pltpu.VMEM((NCH_P, CH_S), jnp.int32),
        [pltpu.VMEM((CH_S, D_FEAT), jnp.float32) for _ in range(NBUF)],
        pltpu.VMEM_SHARED((N_PAD, D_FEAT), jnp.float32),
        [pltpu.SemaphoreType.DMA for _ in range(NBUF)],
        [pltpu.SemaphoreType.DMA for _ in range(NBUF)],
    ],
    compiler_params=pltpu.CompilerParams(use_tc_tiling_on_sc=False),
)
def _spmm_kernel(feat_hbm, row_hbm, col_hbm, out_hbm,
                 rowi_v, coli_v, gbufs, acc_sh, gsems, ssems):
  gbuf0 = gbufs[0]
  c = lax.axis_index("c")
  s = lax.axis_index("s")
  t = c * NS + s

  _fill_rows(gbuf0, CH_S, D_FEAT, 0.0)
  for j in range(RPT // CH_S):
    pltpu.sync_copy(gbuf0, acc_sh.at[pl.ds(s * RPT + j * CH_S, CH_S)])
  plsc.subcore_barrier()

  for p in range(NPIECE):
    pltpu.sync_copy(row_hbm.at[t, p], rowi_v)
    pltpu.sync_copy(col_hbm.at[t, p], coli_v)

    pltpu.async_copy(feat_hbm.at[coli_v.at[0]], gbufs[0], gsems[0])
    pltpu.async_copy(feat_hbm.at[coli_v.at[1]], gbufs[1], gsems[1])

    def step(j, k):
      buf, gs, ss = gbufs[k], gsems[k], ssems[k]
      kn = (k + 2) % NBUF
      pltpu.make_async_copy(feat_hbm.at[coli_v.at[j]], buf, gs).wait()
      pltpu.async_copy(buf, acc_sh.at[rowi_v.at[j]], ss, add=True)

      @pl.when(j >= 1)
      def _():
        pltpu.make_async_copy(gbufs[kn], acc_sh.at[rowi_v.at[j]],
                              ssems[kn]).wait()

      @pl.when(j + 2 < NCH_P)
      def _():
        pltpu.async_copy(feat_hbm.at[coli_v.at[j + 2]], gbufs[kn], gsems[kn])

    def body(j, _):
      for k in range(NBUF):
        @pl.when(j % NBUF == k)
        def _(k=k):
          step(j, k)

      return 0

    lax.fori_loop(0, NCH_P, body, 0)

    pltpu.make_async_copy(gbufs[(NCH_P - 1) % NBUF],
                          acc_sh.at[rowi_v.at[0]],
                          ssems[(NCH_P - 1) % NBUF]).wait()

  plsc.subcore_barrier()

  pltpu.sync_copy(acc_sh.at[pl.ds(s * RPT, RPT)],
                  out_hbm.at[c, pl.ds(s * RPT, RPT)])


def _final_body(p_ref, wh_ref, o_ref):
  rs = wh_ref[0] + wh_ref[1]
  d = jnp.where(rs > 0, lax.rsqrt(rs), jnp.zeros_like(rs))
  o_ref[...] = (p_ref[0] + p_ref[1]) * d[:, 0:1]


def _final_combine(parts, wh):
  n = N_NODES
  bk = 1000
  grid = n // bk
  return pl.pallas_call(
      _final_body,
      out_shape=jax.ShapeDtypeStruct((n, D_FEAT), jnp.float32),
      grid=(grid,),
      in_specs=[
          pl.BlockSpec((NC, bk, D_FEAT), lambda i: (0, i, 0)),
          pl.BlockSpec((NC, bk, HIST_W), lambda i: (0, i, 0)),
      ],
      out_specs=pl.BlockSpec((bk, D_FEAT), lambda i: (i, 0)),
  )(parts, wh)


def kernel(features, edge_index):
  ei = edge_index.astype(jnp.int32)
  row3 = ei[0].reshape(NW, NCHUNK, CHUNK)
  row4 = ei[0].reshape(NW, NPIECE, NCH_P, CH_S)
  col4 = ei[1].reshape(NW, NPIECE, NCH_P, CH_S)
  wh = _hist_kernel(row3)
  fs = _scale_features(wh, features)
  parts = _spmm_kernel(fs, row4, col4)
  return _final_combine(parts, wh)
